# Initial kernel scaffold; baseline (speedup 1.0000x reference)
import jax
import jax.numpy as jnp
from jax.experimental import pallas as pl


def kernel(x, edge_index, W1, b1, gamma1, beta1, W2, b2):
    raise NotImplementedError("write your pallas kernel here")


# speedup vs baseline: 14.5348x; 14.5348x over previous
import functools

import jax
import jax.numpy as jnp
from jax import lax
from jax.experimental import pallas as pl
from jax.experimental.pallas import tpu as pltpu
from jax.experimental.pallas import tpu_sc as plsc

NC = 2
NS = 16
LANES = 16
EDGE_CHUNK = 80
COPY_ROWS = 200

_BN_EPS = 1e-5
_DEG_W = 16


def _sc_mesh():
    return plsc.VectorSubcoreMesh(
        core_axis_name="c", subcore_axis_name="s", num_cores=NC, num_subcores=NS
    )


def _chunk_sweep(s, nchunk, fn):
    jmax = -(-nchunk // NS)
    for j in range(jmax):
        cid = s + j * NS

        @pl.when(cid < nchunk)
        def _():
            fn(pl.multiple_of(cid * COPY_ROWS, 8))


@functools.cache
def _deg_call(n, e):
    ept = e // (NC * NS)
    chunks = ept // EDGE_CHUNK
    nchunk = n // COPY_ROWS

    def body(dst_hbm, out_hbm, ones_v, idx_v, cbuf, acc_sh):
        c = lax.axis_index("c")
        s = lax.axis_index("s")

        def fill(i, carry):
            ones_v[i] = jnp.ones((LANES,), jnp.float32)
            return carry

        lax.fori_loop(0, EDGE_CHUNK, fill, 0)

        def zfill(i, carry):
            cbuf[i] = jnp.zeros((LANES,), jnp.float32)
            return carry

        lax.fori_loop(0, COPY_ROWS, zfill, 0)
        _chunk_sweep(s, nchunk, lambda r0: pltpu.sync_copy(
            cbuf, acc_sh.at[pl.ds(r0, COPY_ROWS)]))
        plsc.subcore_barrier()

        base = (c * NS + s) * ept

        def step(i, carry):
            pltpu.sync_copy(dst_hbm.at[pl.ds(base + i * EDGE_CHUNK, EDGE_CHUNK)], idx_v)
            pltpu.sync_copy(ones_v, acc_sh.at[idx_v], add=True)
            return carry

        lax.fori_loop(0, chunks, step, 0)
        plsc.subcore_barrier()

        def copy_out(r0):
            pltpu.sync_copy(acc_sh.at[pl.ds(r0, COPY_ROWS)], cbuf)
            pltpu.sync_copy(cbuf, out_hbm.at[c, pl.ds(r0, COPY_ROWS)])

        _chunk_sweep(s, nchunk, copy_out)

    return pl.kernel(
        body,
        out_type=jax.ShapeDtypeStruct((NC, n, _DEG_W), jnp.float32),
        mesh=_sc_mesh(),
        scratch_types=[
            pltpu.VMEM((EDGE_CHUNK, _DEG_W), jnp.float32),
            pltpu.VMEM((EDGE_CHUNK,), jnp.int32),
            pltpu.VMEM((COPY_ROWS, _DEG_W), jnp.float32),
            pltpu.VMEM_SHARED((n, _DEG_W), jnp.float32),
        ],
    )


@functools.cache
def _agg_call(n, e, f):
    ept = e // (NC * NS)
    chunks = ept // EDGE_CHUNK
    nchunk = n // COPY_ROWS
    nf = f // LANES

    def body(tab_hbm, src_hbm, dst_hbm, out_hbm, idx_s, idx_d, rows, cbuf, acc_sh, sem):
        c = lax.axis_index("c")
        s = lax.axis_index("s")

        def zfill(i, carry):
            for k in range(nf):
                cbuf[i, pl.ds(k * LANES, LANES)] = jnp.zeros((LANES,), jnp.float32)
            return carry

        lax.fori_loop(0, COPY_ROWS, zfill, 0)
        _chunk_sweep(s, nchunk, lambda r0: pltpu.sync_copy(
            cbuf, acc_sh.at[pl.ds(r0, COPY_ROWS)]))
        plsc.subcore_barrier()

        base = (c * NS + s) * ept

        def step(i, carry):
            pltpu.sync_copy(src_hbm.at[pl.ds(base + i * EDGE_CHUNK, EDGE_CHUNK)], idx_s)
            pltpu.sync_copy(dst_hbm.at[pl.ds(base + i * EDGE_CHUNK, EDGE_CHUNK)], idx_d)
            pltpu.async_copy(tab_hbm.at[idx_s], rows, sem).wait()
            pltpu.sync_copy(rows, acc_sh.at[idx_d], add=True)
            return carry

        lax.fori_loop(0, chunks, step, 0)
        plsc.subcore_barrier()

        def copy_out(r0):
            pltpu.sync_copy(acc_sh.at[pl.ds(r0, COPY_ROWS)], cbuf)
            pltpu.sync_copy(cbuf, out_hbm.at[c, pl.ds(r0, COPY_ROWS)])

        _chunk_sweep(s, nchunk, copy_out)

    return pl.kernel(
        body,
        out_type=jax.ShapeDtypeStruct((NC, n, f), jnp.float32),
        mesh=_sc_mesh(),
        scratch_types=[
            pltpu.VMEM((EDGE_CHUNK,), jnp.int32),
            pltpu.VMEM((EDGE_CHUNK,), jnp.int32),
            pltpu.VMEM((EDGE_CHUNK, f), jnp.float32),
            pltpu.VMEM((COPY_ROWS, f), jnp.float32),
            pltpu.VMEM_SHARED((n, f), jnp.float32),
            pltpu.SemaphoreType.DMA,
        ],
    )


def _prep_body(dp_ref, x_ref, xs_ref, dinv_ref):
    dp = dp_ref[...]
    deg = dp[0, :, 0:1] + dp[1, :, 0:1] + 1.0
    dinv = lax.rsqrt(deg)
    dinv_ref[...] = dinv
    xs_ref[...] = x_ref[...] * dinv


def _mid_body(acc_ref, xs_ref, dinv_ref, w1_ref, b1_ref, g1_ref, be1_ref, w2_ref, ys_ref):
    acc = acc_ref[...]
    dinv = dinv_ref[...]
    z = (acc[0] + acc[1] + xs_ref[...]) * dinv
    h = jnp.dot(z, w1_ref[...], preferred_element_type=jnp.float32) + b1_ref[...]
    mean = jnp.mean(h, axis=0, keepdims=True)
    var = jnp.mean((h - mean) ** 2, axis=0, keepdims=True)
    hn = (h - mean) * lax.rsqrt(var + _BN_EPS) * g1_ref[...] + be1_ref[...]
    hn = jnp.maximum(hn, 0.0)
    y = jnp.dot(hn, w2_ref[...], preferred_element_type=jnp.float32)
    ys_ref[...] = y * dinv


def _fin_body(acc_ref, ys_ref, dinv_ref, b2_ref, out_ref):
    acc = acc_ref[...]
    out_ref[...] = (acc[0] + acc[1] + ys_ref[...]) * dinv_ref[...] + b2_ref[...]


def kernel(x, edge_index, W1, b1, gamma1, beta1, W2, b2):
    n, f = x.shape
    e = edge_index.shape[1]
    src = edge_index[0].astype(jnp.int32)
    dst = edge_index[1].astype(jnp.int32)

    deg_parts = _deg_call(n, e)(dst)
    xs, dinv = pl.pallas_call(
        _prep_body,
        out_shape=[
            jax.ShapeDtypeStruct((n, f), jnp.float32),
            jax.ShapeDtypeStruct((n, 1), jnp.float32),
        ],
    )(deg_parts, x)

    agg = _agg_call(n, e, f)
    acc1 = agg(xs, src, dst)

    ys = pl.pallas_call(
        _mid_body,
        out_shape=jax.ShapeDtypeStruct((n, W2.shape[1]), jnp.float32),
    )(
        acc1, xs, dinv, W1,
        b1.reshape(1, -1), gamma1.reshape(1, -1), beta1.reshape(1, -1), W2,
    )

    acc2 = agg(ys, src, dst)
    out = pl.pallas_call(
        _fin_body,
        out_shape=jax.ShapeDtypeStruct((n, W2.shape[1]), jnp.float32),
    )(acc2, ys, dinv, b2.reshape(1, -1))
    return out
